# baseline (speedup 1.0000x reference)
import jax
import jax.numpy as jnp
from jax import lax
from jax.experimental import pallas as pl
from jax.experimental.pallas import tpu as pltpu

_IMGS = 8


def _make_body(H, W, C, T):
    HW = H * W
    PR = 48

    def body(x_ref, w1_ref, b1_ref, w2_ref, b2_ref, out_ref, *pads):
        for i in range(_IMGS):
            for j, d in ((0, -1), (1, 0), (2, 1)):
                ref = pads[3 * i + j]
                ref[:PR - d, :] = jnp.zeros((PR - d, C), jnp.bfloat16)
                ref[PR - d + HW:, :] = jnp.zeros((PR + d, C), jnp.bfloat16)

        rowid = lax.broadcasted_iota(jnp.int32, (HW, C), 0) % W
        not_last = rowid != (W - 1)
        not_first = rowid != 0
        zero = jnp.zeros((HW, C), jnp.bfloat16)

        def conv3x3(act, wref, t, bref, padm_ref, pad0_ref, padp_ref):
            a_bf = act.T.astype(jnp.bfloat16)
            pad0_ref[PR:PR + HW, :] = a_bf
            padm_ref[PR + 1:PR + 1 + HW, :] = jnp.where(not_last, a_bf, zero)
            padp_ref[PR - 1:PR - 1 + HW, :] = jnp.where(not_first, a_bf, zero)
            taps = []
            for k in range(9):
                oy, ox = k // 3 - 1, k % 3 - 1
                src_ref = (padm_ref, pad0_ref, padp_ref)[ox + 1]
                s = PR + oy * W
                taps.append(src_ref[s:s + HW, :])
            col = jnp.concatenate(taps, axis=1)
            y = lax.dot_general(wref[t], col, (((0,), (1,)), ((), ())),
                                preferred_element_type=jnp.float32)
            return y + bref[t]

        def block(t, carry):
            rs = [out_ref[i] for i in range(_IMGS)]
            y1s = [jnp.maximum(conv3x3(rs[i], w1_ref, t, b1_ref,
                                       *pads[3 * i:3 * i + 3]), 0.0)
                   for i in range(_IMGS)]
            y2s = [conv3x3(y1s[i], w2_ref, t, b2_ref,
                           *pads[3 * i:3 * i + 3])
                   for i in range(_IMGS)]
            for i in range(_IMGS):
                out_ref[i] = rs[i] + y2s[i]
            return carry

        out_ref[...] = x_ref[...]
        for t in range(T):
            block(t, 0)

    return body


def kernel(x, w1, b1, w2, b2):
    N, C, H, W = x.shape
    T = w1.shape[0]
    HW = H * W
    PR = 48

    w1m = w1.astype(jnp.bfloat16).reshape(T, 9 * C, C)
    w2m = w2.astype(jnp.bfloat16).reshape(T, 9 * C, C)
    b1m = jnp.transpose(b1, (0, 2, 1))
    b2m = jnp.transpose(b2, (0, 2, 1))

    xf = x.reshape(N, C, HW)
    out = pl.pallas_call(
        _make_body(H, W, C, T),
        out_shape=jax.ShapeDtypeStruct((N, C, HW), x.dtype),
        grid=(N // _IMGS,),
        in_specs=[
            pl.BlockSpec((_IMGS, C, HW), lambda n: (n, 0, 0)),
            pl.BlockSpec((T, 9 * C, C), lambda n: (0, 0, 0)),
            pl.BlockSpec((T, C, 1), lambda n: (0, 0, 0)),
            pl.BlockSpec((T, 9 * C, C), lambda n: (0, 0, 0)),
            pl.BlockSpec((T, C, 1), lambda n: (0, 0, 0)),
        ],
        out_specs=pl.BlockSpec((_IMGS, C, HW), lambda n: (n, 0, 0)),
        scratch_shapes=[pltpu.VMEM((2 * PR + HW, C), jnp.bfloat16)
                        for _ in range(3 * _IMGS)],
        compiler_params=pltpu.CompilerParams(
            dimension_semantics=("parallel",)),
    )(xf, w1m, b1m, w2m, b2m)
    return out.reshape(N, C, H, W)


# speedup vs baseline: 1.2295x; 1.2295x over previous
import jax
import jax.numpy as jnp
from jax import lax
from jax.experimental import pallas as pl
from jax.experimental.pallas import tpu as pltpu

_IMGS = 4


def _make_body(H, W, C, T):
    HW = H * W
    PR = 48

    def body(x_ref, w1_ref, b1_ref, w2_ref, b2_ref, out_ref, *pads):
        for i in range(_IMGS):
            for j, d in ((0, -1), (1, 0), (2, 1)):
                ref = pads[3 * i + j]
                ref[:PR - d, :] = jnp.zeros((PR - d, C), jnp.bfloat16)
                ref[PR - d + HW:, :] = jnp.zeros((PR + d, C), jnp.bfloat16)

        rowid = lax.broadcasted_iota(jnp.int32, (HW, C), 0) % W
        not_last = rowid != (W - 1)
        not_first = rowid != 0
        zero = jnp.zeros((HW, C), jnp.bfloat16)

        def conv3x3(act, wref, t, bref, padm_ref, pad0_ref, padp_ref):
            a_bf = act.T.astype(jnp.bfloat16)
            pad0_ref[PR:PR + HW, :] = a_bf
            padm_ref[PR + 1:PR + 1 + HW, :] = jnp.where(not_last, a_bf, zero)
            padp_ref[PR - 1:PR - 1 + HW, :] = jnp.where(not_first, a_bf, zero)
            taps = []
            for k in range(9):
                oy, ox = k // 3 - 1, k % 3 - 1
                src_ref = (padm_ref, pad0_ref, padp_ref)[ox + 1]
                s = PR + oy * W
                taps.append(src_ref[s:s + HW, :])
            col = jnp.concatenate(taps, axis=1)
            y = lax.dot_general(wref[t], col, (((0,), (1,)), ((), ())),
                                preferred_element_type=jnp.float32)
            return y + bref[t]

        def block(t):
            rs = [out_ref[i] for i in range(_IMGS)]
            y1s = [jnp.maximum(conv3x3(rs[i], w1_ref, t, b1_ref,
                                       *pads[3 * i:3 * i + 3]), 0.0)
                   for i in range(_IMGS)]
            y2s = [conv3x3(y1s[i], w2_ref, t, b2_ref,
                           *pads[3 * i:3 * i + 3])
                   for i in range(_IMGS)]
            for i in range(_IMGS):
                out_ref[i] = rs[i] + y2s[i]

        out_ref[...] = x_ref[...]
        for t in range(T):
            block(t)

    return body


def kernel(x, w1, b1, w2, b2):
    N, C, H, W = x.shape
    T = w1.shape[0]
    HW = H * W
    PR = 48

    w1m = w1.astype(jnp.bfloat16).reshape(T, 9 * C, C)
    w2m = w2.astype(jnp.bfloat16).reshape(T, 9 * C, C)
    b1m = jnp.transpose(b1, (0, 2, 1))
    b2m = jnp.transpose(b2, (0, 2, 1))

    xf = x.reshape(N, C, HW)
    out = pl.pallas_call(
        _make_body(H, W, C, T),
        out_shape=jax.ShapeDtypeStruct((N, C, HW), x.dtype),
        grid=(N // _IMGS,),
        in_specs=[
            pl.BlockSpec((_IMGS, C, HW), lambda n: (n, 0, 0)),
            pl.BlockSpec((T, 9 * C, C), lambda n: (0, 0, 0)),
            pl.BlockSpec((T, C, 1), lambda n: (0, 0, 0)),
            pl.BlockSpec((T, 9 * C, C), lambda n: (0, 0, 0)),
            pl.BlockSpec((T, C, 1), lambda n: (0, 0, 0)),
        ],
        out_specs=pl.BlockSpec((_IMGS, C, HW), lambda n: (n, 0, 0)),
        scratch_shapes=[pltpu.VMEM((2 * PR + HW, C), jnp.bfloat16)
                        for _ in range(3 * _IMGS)],
        compiler_params=pltpu.CompilerParams(
            dimension_semantics=("parallel",)),
    )(xf, w1m, b1m, w2m, b2m)
    return out.reshape(N, C, H, W)
